# baseline (speedup 1.0000x reference)
import jax
import jax.numpy as jnp
from jax.experimental import pallas as pl
from jax.experimental.pallas import tpu as pltpu

N = 8192
D = 128
C = 2048
K = N // C
INV = 1.0 / (N - 1)


def _body(nodes_ref, w2_ref, b2_ref, w3_ref, b3_ref, w5_ref, b5_ref,
          out_ref, nstash_ref, dn_ref, dcol_ref, obuf_ref, isem, osem):
    for k in range(K):
        rows = pl.ds(k * C, C)
        pltpu.make_async_copy(
            nodes_ref.at[rows, :], nstash_ref.at[rows, :], isem.at[k]
        ).start()

    s = jnp.zeros((D, D), jnp.float32)
    hs = jnp.zeros((1, D), jnp.float32)
    for k in range(K):
        rows = pl.ds(k * C, C)
        pltpu.make_async_copy(
            nodes_ref.at[rows, :], nstash_ref.at[rows, :], isem.at[k]
        ).wait()
        nc = nstash_ref[rows, :]
        ncb = nc.astype(jnp.bfloat16)
        hc = jax.lax.dot_general(
            ncb, w3_ref[:].astype(jnp.bfloat16), (((1,), (1,)), ((), ())),
            preferred_element_type=jnp.float32) + b3_ref[:]
        hcb = hc.astype(jnp.bfloat16)
        s = s + jax.lax.dot_general(
            hcb, ncb, (((0,), (0,)), ((), ())),
            preferred_element_type=jnp.float32)
        hs = hs + jnp.sum(hc, axis=0, keepdims=True)
        hsq = hcb * hcb
        d = jax.lax.dot_general(
            hsq, jnp.ones((D, D), jnp.bfloat16), (((1,), (0,)), ((), ())),
            preferred_element_type=jnp.float32)
        dn_ref[rows, :] = (d * nc).astype(jnp.bfloat16)
        dcol_ref[rows, :] = d.astype(jnp.bfloat16)

    w25 = jax.lax.dot_general(
        w5_ref[:], w2_ref[:], (((1,), (0,)), ((), ())),
        preferred_element_type=jnp.float32)
    c25 = jax.lax.dot_general(
        b2_ref[:], w5_ref[:], (((1,), (1,)), ((), ())),
        preferred_element_type=jnp.float32)
    t = jax.lax.dot_general(
        s, w25, (((1,), (1,)), ((), ())),
        preferred_element_type=jnp.float32) + jax.lax.dot_general(
        hs, c25, (((0,), (0,)), ((), ())),
        preferred_element_type=jnp.float32)
    u = jax.lax.dot_general(
        w3_ref[:], t, (((0,), (0,)), ((), ())),
        preferred_element_type=jnp.float32) * INV
    c = jax.lax.dot_general(
        b3_ref[:], t, (((1,), (0,)), ((), ())),
        preferred_element_type=jnp.float32) * INV + b5_ref[:]
    rhs = jnp.concatenate(
        [u.astype(jnp.bfloat16), (w25 * -INV).T.astype(jnp.bfloat16)],
        axis=0)
    c25i = c25 * INV

    for k in range(K):
        rows = pl.ds(k * C, C)
        nc = nstash_ref[rows, :]
        lhs = jnp.concatenate(
            [nc.astype(jnp.bfloat16), dn_ref[rows, :]], axis=1)
        obuf_ref[rows, :] = jax.lax.dot_general(
            lhs, rhs, (((1,), (0,)), ((), ())),
            preferred_element_type=jnp.float32)
        obuf_ref[rows, :] += nc + c - (
            dcol_ref[rows, :].astype(jnp.float32) * c25i)
        pltpu.make_async_copy(
            obuf_ref.at[rows, :], out_ref.at[rows, :], osem.at[k]
        ).start()

    for k in range(K):
        rows = pl.ds(k * C, C)
        pltpu.make_async_copy(
            obuf_ref.at[rows, :], out_ref.at[rows, :], osem.at[k]
        ).wait()


@jax.jit
def kernel(nodes_in, inputs, W2, b2, W3, b3, W5, b5):
    del inputs
    hbm = pl.BlockSpec(memory_space=pltpu.MemorySpace.HBM)
    full_dd = pl.BlockSpec((D, D), lambda: (0, 0))
    full_1d = pl.BlockSpec((1, D), lambda: (0, 0))

    return pl.pallas_call(
        _body,
        grid=(),
        in_specs=[hbm, full_dd, full_1d, full_dd, full_1d,
                  full_dd, full_1d],
        out_specs=hbm,
        out_shape=jax.ShapeDtypeStruct((N, D), jnp.float32),
        scratch_shapes=[
            pltpu.VMEM((N, D), jnp.float32),
            pltpu.VMEM((N, D), jnp.bfloat16),
            pltpu.VMEM((N, D), jnp.bfloat16),
            pltpu.VMEM((N, D), jnp.float32),
            pltpu.SemaphoreType.DMA((K,)),
            pltpu.SemaphoreType.DMA((K,)),
        ],
    )(nodes_in, W2, b2.reshape(1, D), W3, b3.reshape(1, D),
      W5, b5.reshape(1, D))
